# baseline (speedup 1.0000x reference)
import jax
import jax.numpy as jnp
from jax import lax
from jax.experimental import pallas as pl
from jax.experimental.pallas import tpu as pltpu
from jax.experimental.pallas import tpu_sc as plsc

_N = 10000
_NP = 10240
_E = 320000
_CP = 48
_NC, _NS = 2, 16
_NW = _NC * _NS
_EPT = _E // _NW
_BLK = 80
_NBLK = _EPT // _BLK
_RPT = _NP // _NS
_RB = 1280
_NITER = 10
_EXPL = 0.1


def _prop_body(q_hbm, src_hbm, dst_hbm, out_hbm, src_v, dst_v, rowbuf, acc):
    cid = lax.axis_index("c")
    sid = lax.axis_index("s")
    wid = sid * _NC + cid
    r0 = sid * _RPT
    pltpu.sync_copy(q_hbm.at[pl.ds(r0, _RPT)], acc.at[pl.ds(r0, _RPT)])
    pltpu.sync_copy(src_hbm.at[wid], src_v)
    pltpu.sync_copy(dst_hbm.at[wid], dst_v)
    plsc.subcore_barrier()

    @pl.loop(0, _NBLK)
    def _(b):
        pltpu.sync_copy(q_hbm.at[src_v.at[b]], rowbuf)
        pltpu.sync_copy(rowbuf, acc.at[dst_v.at[b]], add=True)

    plsc.subcore_barrier()
    pltpu.sync_copy(acc.at[pl.ds(r0, _RPT)], out_hbm.at[cid, pl.ds(r0, _RPT)])


def _sc_propagate(q, src_t, dst_t):
    mesh = plsc.VectorSubcoreMesh(core_axis_name="c", subcore_axis_name="s")
    f = pl.kernel(
        _prop_body,
        out_type=jax.ShapeDtypeStruct((_NC, _NP, _CP), jnp.float32),
        mesh=mesh,
        compiler_params=pltpu.CompilerParams(use_tc_tiling_on_sc=False),
        scratch_types=[
            pltpu.VMEM((_NBLK, _BLK), jnp.int32),
            pltpu.VMEM((_NBLK, _BLK), jnp.int32),
            pltpu.VMEM((_BLK, _CP), jnp.float32),
            pltpu.VMEM_SHARED((_NP, _CP), jnp.float32),
        ],
    )
    return f(q, src_t, dst_t)


def _enc_body(x_ref, w1_ref, b1_ref, w2_ref, b2_ref, z_ref):
    h = jnp.dot(x_ref[...], w1_ref[...], preferred_element_type=jnp.float32)
    h = jnp.maximum(h + b1_ref[...], 0.0)
    z_ref[...] = jnp.dot(h, w2_ref[...], preferred_element_type=jnp.float32) + b2_ref[...]


def _tc_encoder(x, W1, b1, W2p, b2p):
    d_in = x.shape[1]
    hid = W1.shape[1]
    return pl.pallas_call(
        _enc_body,
        grid=(_NP // _RB,),
        in_specs=[
            pl.BlockSpec((_RB, d_in), lambda i: (i, 0)),
            pl.BlockSpec((d_in, hid), lambda i: (0, 0)),
            pl.BlockSpec((1, hid), lambda i: (0, 0)),
            pl.BlockSpec((hid, _CP), lambda i: (0, 0)),
            pl.BlockSpec((1, _CP), lambda i: (0, 0)),
        ],
        out_specs=pl.BlockSpec((_RB, _CP), lambda i: (i, 0)),
        out_shape=jax.ShapeDtypeStruct((_NP, _CP), jnp.float32),
    )(x, W1, b1, W2p, b2p)


def _prep_body(s_ref, z_ref, dinv_ref, q_ref):
    deg = s_ref[0, :, 0:1] + s_ref[1, :, 0:1] - 1.0
    dinv = jnp.where(deg > 0, 1.0 / jnp.sqrt(deg), 0.0)
    dinv_ref[...] = dinv
    q_ref[...] = dinv * z_ref[...]


def _tc_prep(s_deg, z):
    return pl.pallas_call(
        _prep_body,
        grid=(_NP // _RB,),
        in_specs=[
            pl.BlockSpec((_NC, _RB, _CP), lambda i: (0, i, 0)),
            pl.BlockSpec((_RB, _CP), lambda i: (i, 0)),
        ],
        out_specs=[
            pl.BlockSpec((_RB, 1), lambda i: (i, 0)),
            pl.BlockSpec((_RB, _CP), lambda i: (i, 0)),
        ],
        out_shape=[
            jax.ShapeDtypeStruct((_NP, 1), jnp.float32),
            jax.ShapeDtypeStruct((_NP, _CP), jnp.float32),
        ],
    )(s_deg, z)


def _heads(xcur, w):
    ph = jnp.maximum(jnp.dot(xcur, w["pW1"][...], preferred_element_type=jnp.float32) + w["pb1"][...], 0.0)
    ph = jnp.maximum(jnp.dot(ph, w["pW2"][...], preferred_element_type=jnp.float32) + w["pb2"][...], 0.0)
    hl = jnp.dot(ph, w["pW3"][...], preferred_element_type=jnp.float32) + w["pb3"][...]
    vh = jnp.maximum(jnp.dot(xcur, w["vW1"][...], preferred_element_type=jnp.float32) + w["vb1"][...], 0.0)
    vh = jnp.maximum(jnp.dot(vh, w["vW2"][...], preferred_element_type=jnp.float32) + w["vb2"][...], 0.0)
    v = jnp.dot(vh, w["vW3"][...], preferred_element_type=jnp.float32) + w["vb3"][...]
    return hl, v


def _iter_body(last, s_ref, prop_ref, q_ref, dinv_ref, act_ref, steps_ref,
               hlp_ref, hv_ref, hent_ref, noise_ref, u_ref, stepv_ref,
               pW1, pb1, pW2, pb2, pW3, pb3, vW1, vb1, vW2, vb2, vW3, vb3,
               *out_refs):
    w = {"pW1": pW1, "pb1": pb1, "pW2": pW2, "pb2": pb2, "pW3": pW3, "pb3": pb3,
         "vW1": vW1, "vb1": vb1, "vW2": vW2, "vb2": vb2, "vW3": vW3, "vb3": vb3}
    dinv = dinv_ref[...]
    active = act_ref[...] > 0.5
    scat = s_ref[0] + s_ref[1] - q_ref[...]
    new_prop = dinv * scat
    xcur = jnp.where(active, new_prop, prop_ref[...])
    hl, v = _heads(xcur, w)
    p = jax.nn.sigmoid(hl)
    entropy = -(p * jnp.log(p + 1e-10) + (1.0 - p) * jnp.log(1.0 - p + 1e-10))
    noisy_p = jnp.clip(p + noise_ref[...], 0.01, 0.99)
    halt = active & (u_ref[...] < noisy_p)
    lnp = jnp.log(noisy_p + 1e-10)
    hlp = jnp.where(halt, lnp, hlp_ref[...])
    hv = jnp.where(halt, v, hv_ref[...])
    hent = jnp.where(halt, entropy, hent_ref[...])
    active2 = active & (~halt)
    steps = jnp.where(active2, stepv_ref[0, 0], steps_ref[...])
    if not last:
        (xcur_ref, qn_ref, act_o, steps_o, hlp_o, hv_o, hent_o) = out_refs
        xcur_ref[...] = xcur
        qn_ref[...] = dinv * xcur
        act_o[...] = jnp.where(active2, 1.0, 0.0)
        steps_o[...] = steps
        hlp_o[...] = hlp
        hv_o[...] = hv
        hent_o[...] = hent
    else:
        (out_ref, steps_o, hlp_o, hv_o, hent_o) = out_refs
        hlp_o[...] = jnp.where(active2, lnp, hlp)
        hv_o[...] = jnp.where(active2, v, hv)
        hent_o[...] = jnp.where(active2, entropy, hent)
        steps_o[...] = jnp.where(active2, float(_NITER), steps)
        logits = xcur[:, :40]
        m = jnp.max(logits, axis=1, keepdims=True)
        sh = logits - m
        out_ref[...] = sh - jnp.log(jnp.sum(jnp.exp(sh), axis=1, keepdims=True))


def _tc_iter(last, s, prop, q, dinv, act, steps, hlp, hv, hent, noise_t, u_t,
             stepv, wts):
    rb = pl.BlockSpec((_RB, _CP), lambda i: (i, 0))
    cb = pl.BlockSpec((_RB, 1), lambda i: (i, 0))
    full = lambda a: pl.BlockSpec(a.shape, lambda i: tuple(0 for _ in a.shape))
    in_specs = [
        pl.BlockSpec((_NC, _RB, _CP), lambda i: (0, i, 0)),
        rb, rb, cb, cb, cb, cb, cb, cb, cb, cb,
        pl.BlockSpec((1, 1), lambda i: (0, 0)),
    ] + [full(w) for w in wts]
    if not last:
        out_specs = [rb, rb, cb, cb, cb, cb, cb]
        out_shape = [
            jax.ShapeDtypeStruct((_NP, _CP), jnp.float32),
            jax.ShapeDtypeStruct((_NP, _CP), jnp.float32),
        ] + [jax.ShapeDtypeStruct((_NP, 1), jnp.float32)] * 5
    else:
        out_specs = [pl.BlockSpec((_RB, 40), lambda i: (i, 0)), cb, cb, cb, cb]
        out_shape = [jax.ShapeDtypeStruct((_NP, 40), jnp.float32)] + \
                    [jax.ShapeDtypeStruct((_NP, 1), jnp.float32)] * 4
    import functools
    return pl.pallas_call(
        functools.partial(_iter_body, last),
        grid=(_NP // _RB,),
        in_specs=in_specs,
        out_specs=out_specs,
        out_shape=out_shape,
    )(s, prop, q, dinv, act, steps, hlp, hv, hent, noise_t, u_t, stepv, *wts)


def kernel(x, edge_index, W1, b1, W2, b2, pW1, pb1, pW2, pb2, pW3, pb3,
           vW1, vb1, vW2, vb2, vW3, vb3):
    f32 = jnp.float32
    W2p = jnp.pad(W2, ((0, 0), (0, _CP - 40)))
    b2p = jnp.pad(b2, (0, _CP - 40)).reshape(1, _CP)
    pW1p = jnp.pad(pW1, ((0, _CP - 40), (0, 0)))
    vW1p = jnp.pad(vW1, ((0, _CP - 40), (0, 0)))
    wts = [pW1p, pb1.reshape(1, -1), pW2, pb2.reshape(1, -1), pW3,
           pb3.reshape(1, 1), vW1p, vb1.reshape(1, -1), vW2, vb2.reshape(1, -1),
           vW3, vb3.reshape(1, 1)]
    src_t = edge_index[0].reshape(_NW, _NBLK, _BLK)
    dst_t = edge_index[1].reshape(_NW, _NBLK, _BLK)

    rkey = jax.random.key(42)
    noise_all = jnp.stack([
        jax.random.normal(jax.random.fold_in(rkey, 2 * t), (_N,), dtype=f32) * _EXPL
        for t in range(_NITER)])
    u_all = jnp.stack([
        jax.random.uniform(jax.random.fold_in(rkey, 2 * t + 1), (_N,), dtype=f32)
        for t in range(_NITER)])
    noise_all = jnp.pad(noise_all, ((0, 0), (0, _NP - _N)))
    u_all = jnp.pad(u_all, ((0, 0), (0, _NP - _N)))
    xp = jnp.pad(x, ((0, _NP - _N), (0, 0)))

    z = _tc_encoder(xp, W1, b1.reshape(1, -1), W2p, b2p)
    s_deg = _sc_propagate(jnp.ones((_NP, _CP), f32), src_t, dst_t)
    dinv, q = _tc_prep(s_deg, z)

    prop = z
    act = jnp.ones((_NP, 1), f32)
    steps = jnp.ones((_NP, 1), f32)
    hlp = jnp.zeros((_NP, 1), f32)
    hv = jnp.zeros((_NP, 1), f32)
    hent = jnp.zeros((_NP, 1), f32)

    for t in range(_NITER):
        s = _sc_propagate(q, src_t, dst_t)
        noise_t = noise_all[t].reshape(_NP, 1)
        u_t = u_all[t].reshape(_NP, 1)
        stepv = jnp.full((1, 1), float(t + 2), f32)
        last = t == _NITER - 1
        outs = _tc_iter(last, s, prop, q, dinv, act, steps, hlp, hv, hent,
                        noise_t, u_t, stepv, wts)
        if not last:
            prop, q, act, steps, hlp, hv, hent = outs
        else:
            out, steps, hlp, hv, hent = outs

    return (out[:_N], steps[:_N].reshape(_N), hlp[:_N].reshape(_N),
            hv[:_N].reshape(_N), hent[:_N].reshape(_N))


# speedup vs baseline: 1.7745x; 1.7745x over previous
import jax
import jax.numpy as jnp
from jax import lax
from jax.experimental import pallas as pl
from jax.experimental.pallas import tpu as pltpu
from jax.experimental.pallas import tpu_sc as plsc

_N = 10000
_NP = 10240
_E = 320000
_CP = 48
_NC, _NS = 2, 16
_NW = _NC * _NS
_EPT = _E // _NW
_BLK = 125
_NBLK = _EPT // _BLK
_K = 8
_NCHUNK = _NBLK // _K
_RPT = _NP // _NS
_RB = 1280
_NITER = 10
_EXPL = 0.1


def _prop_body(q_hbm, src_hbm, dst_hbm, out_hbm, src_v, dst_v, rowbuf,
               gsem, ssem, acc):
    cid = lax.axis_index("c")
    sid = lax.axis_index("s")
    wid = sid * _NC + cid
    r0 = sid * _RPT
    pltpu.sync_copy(q_hbm.at[pl.ds(r0, _RPT)], acc.at[pl.ds(r0, _RPT)])
    pltpu.sync_copy(src_hbm.at[wid], src_v)
    pltpu.sync_copy(dst_hbm.at[wid], dst_v)
    plsc.subcore_barrier()

    @pl.loop(0, _NCHUNK)
    def _(c):
        ghs = [pltpu.async_copy(q_hbm.at[src_v.at[c * _K + j]],
                                rowbuf.at[pl.ds(j * _BLK, _BLK)], gsem)
               for j in range(_K)]
        shs = []
        for j in range(_K):
            ghs[j].wait()
            shs.append(pltpu.async_copy(rowbuf.at[pl.ds(j * _BLK, _BLK)],
                                        acc.at[dst_v.at[c * _K + j]], ssem,
                                        add=True))
        for j in range(_K):
            shs[j].wait()

    plsc.subcore_barrier()
    pltpu.sync_copy(acc.at[pl.ds(r0, _RPT)], out_hbm.at[cid, pl.ds(r0, _RPT)])


def _sc_propagate(q, src_t, dst_t):
    mesh = plsc.VectorSubcoreMesh(core_axis_name="c", subcore_axis_name="s")
    f = pl.kernel(
        _prop_body,
        out_type=jax.ShapeDtypeStruct((_NC, _NP, _CP), jnp.float32),
        mesh=mesh,
        compiler_params=pltpu.CompilerParams(use_tc_tiling_on_sc=False),
        scratch_types=[
            pltpu.VMEM((_NBLK, _BLK), jnp.int32),
            pltpu.VMEM((_NBLK, _BLK), jnp.int32),
            pltpu.VMEM((_K * _BLK, _CP), jnp.float32),
            pltpu.SemaphoreType.DMA,
            pltpu.SemaphoreType.DMA,
            pltpu.VMEM_SHARED((_NP, _CP), jnp.float32),
        ],
    )
    return f(q, src_t, dst_t)


def _enc_body(x_ref, w1_ref, b1_ref, w2_ref, b2_ref, z_ref):
    h = jnp.dot(x_ref[...], w1_ref[...], preferred_element_type=jnp.float32)
    h = jnp.maximum(h + b1_ref[...], 0.0)
    z_ref[...] = jnp.dot(h, w2_ref[...], preferred_element_type=jnp.float32) + b2_ref[...]


def _tc_encoder(x, W1, b1, W2p, b2p):
    d_in = x.shape[1]
    hid = W1.shape[1]
    return pl.pallas_call(
        _enc_body,
        grid=(_NP // _RB,),
        in_specs=[
            pl.BlockSpec((_RB, d_in), lambda i: (i, 0)),
            pl.BlockSpec((d_in, hid), lambda i: (0, 0)),
            pl.BlockSpec((1, hid), lambda i: (0, 0)),
            pl.BlockSpec((hid, _CP), lambda i: (0, 0)),
            pl.BlockSpec((1, _CP), lambda i: (0, 0)),
        ],
        out_specs=pl.BlockSpec((_RB, _CP), lambda i: (i, 0)),
        out_shape=jax.ShapeDtypeStruct((_NP, _CP), jnp.float32),
    )(x, W1, b1, W2p, b2p)


def _prep_body(s_ref, z_ref, dinv_ref, q_ref):
    deg = s_ref[0, :, 0:1] + s_ref[1, :, 0:1] - 1.0
    dinv = jnp.where(deg > 0, 1.0 / jnp.sqrt(deg), 0.0)
    dinv_ref[...] = dinv
    q_ref[...] = dinv * z_ref[...]


def _tc_prep(s_deg, z):
    return pl.pallas_call(
        _prep_body,
        grid=(_NP // _RB,),
        in_specs=[
            pl.BlockSpec((_NC, _RB, _CP), lambda i: (0, i, 0)),
            pl.BlockSpec((_RB, _CP), lambda i: (i, 0)),
        ],
        out_specs=[
            pl.BlockSpec((_RB, 1), lambda i: (i, 0)),
            pl.BlockSpec((_RB, _CP), lambda i: (i, 0)),
        ],
        out_shape=[
            jax.ShapeDtypeStruct((_NP, 1), jnp.float32),
            jax.ShapeDtypeStruct((_NP, _CP), jnp.float32),
        ],
    )(s_deg, z)


def _heads(xcur, w):
    ph = jnp.maximum(jnp.dot(xcur, w["pW1"][...], preferred_element_type=jnp.float32) + w["pb1"][...], 0.0)
    ph = jnp.maximum(jnp.dot(ph, w["pW2"][...], preferred_element_type=jnp.float32) + w["pb2"][...], 0.0)
    hl = jnp.dot(ph, w["pW3"][...], preferred_element_type=jnp.float32) + w["pb3"][...]
    vh = jnp.maximum(jnp.dot(xcur, w["vW1"][...], preferred_element_type=jnp.float32) + w["vb1"][...], 0.0)
    vh = jnp.maximum(jnp.dot(vh, w["vW2"][...], preferred_element_type=jnp.float32) + w["vb2"][...], 0.0)
    v = jnp.dot(vh, w["vW3"][...], preferred_element_type=jnp.float32) + w["vb3"][...]
    return hl, v


def _iter_body(last, s_ref, prop_ref, q_ref, dinv_ref, act_ref, steps_ref,
               hlp_ref, hv_ref, hent_ref, noise_ref, u_ref, stepv_ref,
               pW1, pb1, pW2, pb2, pW3, pb3, vW1, vb1, vW2, vb2, vW3, vb3,
               *out_refs):
    w = {"pW1": pW1, "pb1": pb1, "pW2": pW2, "pb2": pb2, "pW3": pW3, "pb3": pb3,
         "vW1": vW1, "vb1": vb1, "vW2": vW2, "vb2": vb2, "vW3": vW3, "vb3": vb3}
    dinv = dinv_ref[...]
    active = act_ref[...] > 0.5
    scat = s_ref[0] + s_ref[1] - q_ref[...]
    new_prop = dinv * scat
    xcur = jnp.where(active, new_prop, prop_ref[...])
    hl, v = _heads(xcur, w)
    p = jax.nn.sigmoid(hl)
    entropy = -(p * jnp.log(p + 1e-10) + (1.0 - p) * jnp.log(1.0 - p + 1e-10))
    noisy_p = jnp.clip(p + noise_ref[...], 0.01, 0.99)
    halt = active & (u_ref[...] < noisy_p)
    lnp = jnp.log(noisy_p + 1e-10)
    hlp = jnp.where(halt, lnp, hlp_ref[...])
    hv = jnp.where(halt, v, hv_ref[...])
    hent = jnp.where(halt, entropy, hent_ref[...])
    active2 = active & (~halt)
    steps = jnp.where(active2, stepv_ref[0, 0], steps_ref[...])
    if not last:
        (xcur_ref, qn_ref, act_o, steps_o, hlp_o, hv_o, hent_o) = out_refs
        xcur_ref[...] = xcur
        qn_ref[...] = dinv * xcur
        act_o[...] = jnp.where(active2, 1.0, 0.0)
        steps_o[...] = steps
        hlp_o[...] = hlp
        hv_o[...] = hv
        hent_o[...] = hent
    else:
        (out_ref, steps_o, hlp_o, hv_o, hent_o) = out_refs
        hlp_o[...] = jnp.where(active2, lnp, hlp)
        hv_o[...] = jnp.where(active2, v, hv)
        hent_o[...] = jnp.where(active2, entropy, hent)
        steps_o[...] = jnp.where(active2, float(_NITER), steps)
        logits = xcur[:, :40]
        m = jnp.max(logits, axis=1, keepdims=True)
        sh = logits - m
        out_ref[...] = sh - jnp.log(jnp.sum(jnp.exp(sh), axis=1, keepdims=True))


def _tc_iter(last, s, prop, q, dinv, act, steps, hlp, hv, hent, noise_t, u_t,
             stepv, wts):
    rb = pl.BlockSpec((_RB, _CP), lambda i: (i, 0))
    cb = pl.BlockSpec((_RB, 1), lambda i: (i, 0))
    full = lambda a: pl.BlockSpec(a.shape, lambda i: tuple(0 for _ in a.shape))
    in_specs = [
        pl.BlockSpec((_NC, _RB, _CP), lambda i: (0, i, 0)),
        rb, rb, cb, cb, cb, cb, cb, cb, cb, cb,
        pl.BlockSpec((1, 1), lambda i: (0, 0)),
    ] + [full(w) for w in wts]
    if not last:
        out_specs = [rb, rb, cb, cb, cb, cb, cb]
        out_shape = [
            jax.ShapeDtypeStruct((_NP, _CP), jnp.float32),
            jax.ShapeDtypeStruct((_NP, _CP), jnp.float32),
        ] + [jax.ShapeDtypeStruct((_NP, 1), jnp.float32)] * 5
    else:
        out_specs = [pl.BlockSpec((_RB, 40), lambda i: (i, 0)), cb, cb, cb, cb]
        out_shape = [jax.ShapeDtypeStruct((_NP, 40), jnp.float32)] + \
                    [jax.ShapeDtypeStruct((_NP, 1), jnp.float32)] * 4
    import functools
    return pl.pallas_call(
        functools.partial(_iter_body, last),
        grid=(_NP // _RB,),
        in_specs=in_specs,
        out_specs=out_specs,
        out_shape=out_shape,
    )(s, prop, q, dinv, act, steps, hlp, hv, hent, noise_t, u_t, stepv, *wts)


def kernel(x, edge_index, W1, b1, W2, b2, pW1, pb1, pW2, pb2, pW3, pb3,
           vW1, vb1, vW2, vb2, vW3, vb3):
    f32 = jnp.float32
    W2p = jnp.pad(W2, ((0, 0), (0, _CP - 40)))
    b2p = jnp.pad(b2, (0, _CP - 40)).reshape(1, _CP)
    pW1p = jnp.pad(pW1, ((0, _CP - 40), (0, 0)))
    vW1p = jnp.pad(vW1, ((0, _CP - 40), (0, 0)))
    wts = [pW1p, pb1.reshape(1, -1), pW2, pb2.reshape(1, -1), pW3,
           pb3.reshape(1, 1), vW1p, vb1.reshape(1, -1), vW2, vb2.reshape(1, -1),
           vW3, vb3.reshape(1, 1)]
    src_t = edge_index[0].reshape(_NW, _NBLK, _BLK)
    dst_t = edge_index[1].reshape(_NW, _NBLK, _BLK)

    rkey = jax.random.key(42)
    noise_all = jnp.stack([
        jax.random.normal(jax.random.fold_in(rkey, 2 * t), (_N,), dtype=f32) * _EXPL
        for t in range(_NITER)])
    u_all = jnp.stack([
        jax.random.uniform(jax.random.fold_in(rkey, 2 * t + 1), (_N,), dtype=f32)
        for t in range(_NITER)])
    noise_all = jnp.pad(noise_all, ((0, 0), (0, _NP - _N)))
    u_all = jnp.pad(u_all, ((0, 0), (0, _NP - _N)))
    xp = jnp.pad(x, ((0, _NP - _N), (0, 0)))

    z = _tc_encoder(xp, W1, b1.reshape(1, -1), W2p, b2p)
    s_deg = _sc_propagate(jnp.ones((_NP, _CP), f32), src_t, dst_t)
    dinv, q = _tc_prep(s_deg, z)

    prop = z
    act = jnp.ones((_NP, 1), f32)
    steps = jnp.ones((_NP, 1), f32)
    hlp = jnp.zeros((_NP, 1), f32)
    hv = jnp.zeros((_NP, 1), f32)
    hent = jnp.zeros((_NP, 1), f32)

    for t in range(_NITER):
        s = _sc_propagate(q, src_t, dst_t)
        noise_t = noise_all[t].reshape(_NP, 1)
        u_t = u_all[t].reshape(_NP, 1)
        stepv = jnp.full((1, 1), float(t + 2), f32)
        last = t == _NITER - 1
        outs = _tc_iter(last, s, prop, q, dinv, act, steps, hlp, hv, hent,
                        noise_t, u_t, stepv, wts)
        if not last:
            prop, q, act, steps, hlp, hv, hent = outs
        else:
            out, steps, hlp, hv, hent = outs

    return (out[:_N], steps[:_N].reshape(_N), hlp[:_N].reshape(_N),
            hv[:_N].reshape(_N), hent[:_N].reshape(_N))


# speedup vs baseline: 1.7776x; 1.0017x over previous
import jax
import jax.numpy as jnp
from jax import lax
from jax.experimental import pallas as pl
from jax.experimental.pallas import tpu as pltpu
from jax.experimental.pallas import tpu_sc as plsc

_N = 10000
_NP = 10240
_E = 320000
_CP = 48
_NC, _NS = 2, 16
_NW = _NC * _NS
_EPT = _E // _NW
_BLK = 125
_NBLK = _EPT // _BLK
_K = 8
_NCHUNK = _NBLK // _K
_RPT = _NP // _NS
_RB = 1280
_NITER = 10
_EXPL = 0.1


def _prop_body(q_hbm, src_hbm, dst_hbm, out_hbm, src_v, dst_v, rowbuf,
               gsem, ssem, acc):
    cid = lax.axis_index("c")
    sid = lax.axis_index("s")
    wid = sid * _NC + cid
    r0 = sid * _RPT
    pltpu.sync_copy(q_hbm.at[pl.ds(r0, _RPT)], acc.at[pl.ds(r0, _RPT)])
    pltpu.sync_copy(src_hbm.at[wid], src_v)
    pltpu.sync_copy(dst_hbm.at[wid], dst_v)
    plsc.subcore_barrier()

    @pl.loop(0, _NCHUNK)
    def _(c):
        ghs = [pltpu.async_copy(q_hbm.at[src_v.at[c * _K + j]],
                                rowbuf.at[pl.ds(j * _BLK, _BLK)], gsem)
               for j in range(_K)]
        shs = []
        for j in range(_K):
            ghs[j].wait()
            shs.append(pltpu.async_copy(rowbuf.at[pl.ds(j * _BLK, _BLK)],
                                        acc.at[dst_v.at[c * _K + j]], ssem,
                                        add=True))
        for j in range(_K):
            shs[j].wait()

    plsc.subcore_barrier()
    pltpu.sync_copy(acc.at[pl.ds(r0, _RPT)], out_hbm.at[cid, pl.ds(r0, _RPT)])


def _sc_propagate(q, src_t, dst_t):
    mesh = plsc.VectorSubcoreMesh(core_axis_name="c", subcore_axis_name="s")
    f = pl.kernel(
        _prop_body,
        out_type=jax.ShapeDtypeStruct((_NC, _NP, _CP), jnp.float32),
        mesh=mesh,
        compiler_params=pltpu.CompilerParams(use_tc_tiling_on_sc=False),
        scratch_types=[
            pltpu.VMEM((_NBLK, _BLK), jnp.int32),
            pltpu.VMEM((_NBLK, _BLK), jnp.int32),
            pltpu.VMEM((_K * _BLK, _CP), jnp.float32),
            pltpu.SemaphoreType.DMA,
            pltpu.SemaphoreType.DMA,
            pltpu.VMEM_SHARED((_NP, _CP), jnp.float32),
        ],
    )
    return f(q, src_t, dst_t)


def _enc_body(x_ref, w1_ref, b1_ref, w2_ref, b2_ref, z_ref):
    h = jnp.dot(x_ref[...], w1_ref[...], preferred_element_type=jnp.float32)
    h = jnp.maximum(h + b1_ref[...], 0.0)
    z_ref[...] = jnp.dot(h, w2_ref[...], preferred_element_type=jnp.float32) + b2_ref[...]


def _tc_encoder(x, W1, b1, W2p, b2p):
    d_in = x.shape[1]
    hid = W1.shape[1]
    return pl.pallas_call(
        _enc_body,
        grid=(_NP // _RB,),
        in_specs=[
            pl.BlockSpec((_RB, d_in), lambda i: (i, 0)),
            pl.BlockSpec((d_in, hid), lambda i: (0, 0)),
            pl.BlockSpec((1, hid), lambda i: (0, 0)),
            pl.BlockSpec((hid, _CP), lambda i: (0, 0)),
            pl.BlockSpec((1, _CP), lambda i: (0, 0)),
        ],
        out_specs=pl.BlockSpec((_RB, _CP), lambda i: (i, 0)),
        out_shape=jax.ShapeDtypeStruct((_NP, _CP), jnp.float32),
    )(x, W1, b1, W2p, b2p)


def _prep_body(s_ref, z_ref, dinv_ref, q_ref):
    deg = s_ref[0, :, 0:1] + s_ref[1, :, 0:1] - 1.0
    dinv = jnp.where(deg > 0, 1.0 / jnp.sqrt(deg), 0.0)
    dinv_ref[...] = dinv
    q_ref[...] = dinv * z_ref[...]


def _tc_prep(s_deg, z):
    return pl.pallas_call(
        _prep_body,
        grid=(_NP // _RB,),
        in_specs=[
            pl.BlockSpec((_NC, _RB, _CP), lambda i: (0, i, 0)),
            pl.BlockSpec((_RB, _CP), lambda i: (i, 0)),
        ],
        out_specs=[
            pl.BlockSpec((_RB, 1), lambda i: (i, 0)),
            pl.BlockSpec((_RB, _CP), lambda i: (i, 0)),
        ],
        out_shape=[
            jax.ShapeDtypeStruct((_NP, 1), jnp.float32),
            jax.ShapeDtypeStruct((_NP, _CP), jnp.float32),
        ],
    )(s_deg, z)


def _heads(xcur, w):
    ph = jnp.maximum(jnp.dot(xcur, w["pW1"][...], preferred_element_type=jnp.float32) + w["pb1"][...], 0.0)
    ph = jnp.maximum(jnp.dot(ph, w["pW2"][...], preferred_element_type=jnp.float32) + w["pb2"][...], 0.0)
    hl = jnp.dot(ph, w["pW3"][...], preferred_element_type=jnp.float32) + w["pb3"][...]
    vh = jnp.maximum(jnp.dot(xcur, w["vW1"][...], preferred_element_type=jnp.float32) + w["vb1"][...], 0.0)
    vh = jnp.maximum(jnp.dot(vh, w["vW2"][...], preferred_element_type=jnp.float32) + w["vb2"][...], 0.0)
    v = jnp.dot(vh, w["vW3"][...], preferred_element_type=jnp.float32) + w["vb3"][...]
    return hl, v


def _mask_body(s_ref, q_ref, prop_ref, dinv_ref, act_ref, propn_ref, qn_ref):
    scat = s_ref[0] + s_ref[1] - q_ref[...]
    dinv = dinv_ref[...]
    xcur = jnp.where(act_ref[...] > 0.5, dinv * scat, prop_ref[...])
    propn_ref[...] = xcur
    qn_ref[...] = dinv * xcur


def _tc_mask(s, q, prop, dinv, act):
    rb = pl.BlockSpec((_RB, _CP), lambda i: (i, 0))
    cb = pl.BlockSpec((_RB, 1), lambda i: (i, 0))
    return pl.pallas_call(
        _mask_body,
        grid=(_NP // _RB,),
        in_specs=[pl.BlockSpec((_NC, _RB, _CP), lambda i: (0, i, 0)),
                  rb, rb, cb, cb],
        out_specs=[rb, rb],
        out_shape=[jax.ShapeDtypeStruct((_NP, _CP), jnp.float32)] * 2,
    )(s, q, prop, dinv, act)


def _heads_body(last, prop_ref, act_ref, steps_ref, hlp_ref, hv_ref, hent_ref,
                noise_ref, u_ref, stepv_ref,
                pW1, pb1, pW2, pb2, pW3, pb3, vW1, vb1, vW2, vb2, vW3, vb3,
                *out_refs):
    w = {"pW1": pW1, "pb1": pb1, "pW2": pW2, "pb2": pb2, "pW3": pW3, "pb3": pb3,
         "vW1": vW1, "vb1": vb1, "vW2": vW2, "vb2": vb2, "vW3": vW3, "vb3": vb3}
    active = act_ref[...] > 0.5
    xcur = prop_ref[...]
    hl, v = _heads(xcur, w)
    p = jax.nn.sigmoid(hl)
    entropy = -(p * jnp.log(p + 1e-10) + (1.0 - p) * jnp.log(1.0 - p + 1e-10))
    noisy_p = jnp.clip(p + noise_ref[...], 0.01, 0.99)
    halt = active & (u_ref[...] < noisy_p)
    lnp = jnp.log(noisy_p + 1e-10)
    hlp = jnp.where(halt, lnp, hlp_ref[...])
    hv = jnp.where(halt, v, hv_ref[...])
    hent = jnp.where(halt, entropy, hent_ref[...])
    active2 = active & (~halt)
    steps = jnp.where(active2, stepv_ref[0, 0], steps_ref[...])
    if not last:
        (act_o, steps_o, hlp_o, hv_o, hent_o) = out_refs
        act_o[...] = jnp.where(active2, 1.0, 0.0)
        steps_o[...] = steps
        hlp_o[...] = hlp
        hv_o[...] = hv
        hent_o[...] = hent
    else:
        (out_ref, steps_o, hlp_o, hv_o, hent_o) = out_refs
        hlp_o[...] = jnp.where(active2, lnp, hlp)
        hv_o[...] = jnp.where(active2, v, hv)
        hent_o[...] = jnp.where(active2, entropy, hent)
        steps_o[...] = jnp.where(active2, float(_NITER), steps)
        logits = xcur[:, :40]
        m = jnp.max(logits, axis=1, keepdims=True)
        sh = logits - m
        out_ref[...] = sh - jnp.log(jnp.sum(jnp.exp(sh), axis=1, keepdims=True))


def _tc_heads(last, prop, act, steps, hlp, hv, hent, noise_t, u_t, stepv, wts):
    rb = pl.BlockSpec((_RB, _CP), lambda i: (i, 0))
    cb = pl.BlockSpec((_RB, 1), lambda i: (i, 0))
    full = lambda a: pl.BlockSpec(a.shape, lambda i: tuple(0 for _ in a.shape))
    in_specs = [rb, cb, cb, cb, cb, cb, cb, cb,
                pl.BlockSpec((1, 1), lambda i: (0, 0))] + [full(w) for w in wts]
    if not last:
        out_specs = [cb] * 5
        out_shape = [jax.ShapeDtypeStruct((_NP, 1), jnp.float32)] * 5
    else:
        out_specs = [pl.BlockSpec((_RB, 40), lambda i: (i, 0))] + [cb] * 4
        out_shape = [jax.ShapeDtypeStruct((_NP, 40), jnp.float32)] + \
                    [jax.ShapeDtypeStruct((_NP, 1), jnp.float32)] * 4
    import functools
    return pl.pallas_call(
        functools.partial(_heads_body, last),
        grid=(_NP // _RB,),
        in_specs=in_specs,
        out_specs=out_specs,
        out_shape=out_shape,
    )(prop, act, steps, hlp, hv, hent, noise_t, u_t, stepv, *wts)


def kernel(x, edge_index, W1, b1, W2, b2, pW1, pb1, pW2, pb2, pW3, pb3,
           vW1, vb1, vW2, vb2, vW3, vb3):
    f32 = jnp.float32
    W2p = jnp.pad(W2, ((0, 0), (0, _CP - 40)))
    b2p = jnp.pad(b2, (0, _CP - 40)).reshape(1, _CP)
    pW1p = jnp.pad(pW1, ((0, _CP - 40), (0, 0)))
    vW1p = jnp.pad(vW1, ((0, _CP - 40), (0, 0)))
    wts = [pW1p, pb1.reshape(1, -1), pW2, pb2.reshape(1, -1), pW3,
           pb3.reshape(1, 1), vW1p, vb1.reshape(1, -1), vW2, vb2.reshape(1, -1),
           vW3, vb3.reshape(1, 1)]
    src_t = edge_index[0].reshape(_NW, _NBLK, _BLK)
    dst_t = edge_index[1].reshape(_NW, _NBLK, _BLK)

    rkey = jax.random.key(42)
    noise_all = jnp.stack([
        jax.random.normal(jax.random.fold_in(rkey, 2 * t), (_N,), dtype=f32) * _EXPL
        for t in range(_NITER)])
    u_all = jnp.stack([
        jax.random.uniform(jax.random.fold_in(rkey, 2 * t + 1), (_N,), dtype=f32)
        for t in range(_NITER)])
    noise_all = jnp.pad(noise_all, ((0, 0), (0, _NP - _N)))
    u_all = jnp.pad(u_all, ((0, 0), (0, _NP - _N)))
    xp = jnp.pad(x, ((0, _NP - _N), (0, 0)))

    z = _tc_encoder(xp, W1, b1.reshape(1, -1), W2p, b2p)
    s_deg = _sc_propagate(jnp.ones((_NP, _CP), f32), src_t, dst_t)
    dinv, q = _tc_prep(s_deg, z)

    prop = z
    act = jnp.ones((_NP, 1), f32)
    steps = jnp.ones((_NP, 1), f32)
    hlp = jnp.zeros((_NP, 1), f32)
    hv = jnp.zeros((_NP, 1), f32)
    hent = jnp.zeros((_NP, 1), f32)

    for t in range(_NITER):
        s = _sc_propagate(q, src_t, dst_t)
        prop, q = _tc_mask(s, q, prop, dinv, act)
        noise_t = noise_all[t].reshape(_NP, 1)
        u_t = u_all[t].reshape(_NP, 1)
        stepv = jnp.full((1, 1), float(t + 2), f32)
        last = t == _NITER - 1
        outs = _tc_heads(last, prop, act, steps, hlp, hv, hent,
                         noise_t, u_t, stepv, wts)
        if not last:
            act, steps, hlp, hv, hent = outs
        else:
            out, steps, hlp, hv, hent = outs

    return (out[:_N], steps[:_N].reshape(_N), hlp[:_N].reshape(_N),
            hv[:_N].reshape(_N), hent[:_N].reshape(_N))


# speedup vs baseline: 1.8139x; 1.0204x over previous
import jax
import jax.numpy as jnp
from jax import lax
from jax.experimental import pallas as pl
from jax.experimental.pallas import tpu as pltpu
from jax.experimental.pallas import tpu_sc as plsc

_N = 10000
_NP = 10240
_E = 320000
_CP = 40
_NC, _NS = 2, 16
_NW = _NC * _NS
_EPT = _E // _NW
_BLK = 125
_NBLK = _EPT // _BLK
_K = 10
_NCHUNK = _NBLK // _K
_RPT = _NP // _NS
_RB = 1280
_NITER = 10
_EXPL = 0.1


def _prop_body(q_hbm, src_hbm, dst_hbm, out_hbm, src_v, dst_v, rowbuf,
               gsem, ssem, acc):
    cid = lax.axis_index("c")
    sid = lax.axis_index("s")
    wid = sid * _NC + cid
    r0 = sid * _RPT
    pltpu.sync_copy(q_hbm.at[pl.ds(r0, _RPT)], acc.at[pl.ds(r0, _RPT)])
    pltpu.sync_copy(src_hbm.at[wid], src_v)
    pltpu.sync_copy(dst_hbm.at[wid], dst_v)
    plsc.subcore_barrier()

    @pl.loop(0, _NCHUNK)
    def _(c):
        ghs = [pltpu.async_copy(q_hbm.at[src_v.at[c * _K + j]],
                                rowbuf.at[pl.ds(j * _BLK, _BLK)], gsem)
               for j in range(_K)]
        shs = []
        for j in range(_K):
            ghs[j].wait()
            shs.append(pltpu.async_copy(rowbuf.at[pl.ds(j * _BLK, _BLK)],
                                        acc.at[dst_v.at[c * _K + j]], ssem,
                                        add=True))
        for j in range(_K):
            shs[j].wait()

    plsc.subcore_barrier()
    pltpu.sync_copy(acc.at[pl.ds(r0, _RPT)], out_hbm.at[cid, pl.ds(r0, _RPT)])


def _sc_propagate(q, src_t, dst_t):
    mesh = plsc.VectorSubcoreMesh(core_axis_name="c", subcore_axis_name="s")
    f = pl.kernel(
        _prop_body,
        out_type=jax.ShapeDtypeStruct((_NC, _NP, _CP), jnp.float32),
        mesh=mesh,
        compiler_params=pltpu.CompilerParams(use_tc_tiling_on_sc=False),
        scratch_types=[
            pltpu.VMEM((_NBLK, _BLK), jnp.int32),
            pltpu.VMEM((_NBLK, _BLK), jnp.int32),
            pltpu.VMEM((_K * _BLK, _CP), jnp.float32),
            pltpu.SemaphoreType.DMA,
            pltpu.SemaphoreType.DMA,
            pltpu.VMEM_SHARED((_NP, _CP), jnp.float32),
        ],
    )
    return f(q, src_t, dst_t)


def _enc_body(x_ref, w1_ref, b1_ref, w2_ref, b2_ref, z_ref):
    h = jnp.dot(x_ref[...], w1_ref[...], preferred_element_type=jnp.float32)
    h = jnp.maximum(h + b1_ref[...], 0.0)
    z_ref[...] = jnp.dot(h, w2_ref[...], preferred_element_type=jnp.float32) + b2_ref[...]


def _tc_encoder(x, W1, b1, W2p, b2p):
    d_in = x.shape[1]
    hid = W1.shape[1]
    return pl.pallas_call(
        _enc_body,
        grid=(_NP // _RB,),
        in_specs=[
            pl.BlockSpec((_RB, d_in), lambda i: (i, 0)),
            pl.BlockSpec((d_in, hid), lambda i: (0, 0)),
            pl.BlockSpec((1, hid), lambda i: (0, 0)),
            pl.BlockSpec((hid, _CP), lambda i: (0, 0)),
            pl.BlockSpec((1, _CP), lambda i: (0, 0)),
        ],
        out_specs=pl.BlockSpec((_RB, _CP), lambda i: (i, 0)),
        out_shape=jax.ShapeDtypeStruct((_NP, _CP), jnp.float32),
    )(x, W1, b1, W2p, b2p)


def _prep_body(s_ref, z_ref, dinv_ref, q_ref):
    deg = s_ref[0, :, 0:1] + s_ref[1, :, 0:1] - 1.0
    dinv = jnp.where(deg > 0, 1.0 / jnp.sqrt(deg), 0.0)
    dinv_ref[...] = dinv
    q_ref[...] = dinv * z_ref[...]


def _tc_prep(s_deg, z):
    return pl.pallas_call(
        _prep_body,
        grid=(_NP // _RB,),
        in_specs=[
            pl.BlockSpec((_NC, _RB, _CP), lambda i: (0, i, 0)),
            pl.BlockSpec((_RB, _CP), lambda i: (i, 0)),
        ],
        out_specs=[
            pl.BlockSpec((_RB, 1), lambda i: (i, 0)),
            pl.BlockSpec((_RB, _CP), lambda i: (i, 0)),
        ],
        out_shape=[
            jax.ShapeDtypeStruct((_NP, 1), jnp.float32),
            jax.ShapeDtypeStruct((_NP, _CP), jnp.float32),
        ],
    )(s_deg, z)


def _heads(xcur, w):
    ph = jnp.maximum(jnp.dot(xcur, w["pW1"][...], preferred_element_type=jnp.float32) + w["pb1"][...], 0.0)
    ph = jnp.maximum(jnp.dot(ph, w["pW2"][...], preferred_element_type=jnp.float32) + w["pb2"][...], 0.0)
    hl = jnp.dot(ph, w["pW3"][...], preferred_element_type=jnp.float32) + w["pb3"][...]
    vh = jnp.maximum(jnp.dot(xcur, w["vW1"][...], preferred_element_type=jnp.float32) + w["vb1"][...], 0.0)
    vh = jnp.maximum(jnp.dot(vh, w["vW2"][...], preferred_element_type=jnp.float32) + w["vb2"][...], 0.0)
    v = jnp.dot(vh, w["vW3"][...], preferred_element_type=jnp.float32) + w["vb3"][...]
    return hl, v


def _mask_body(s_ref, q_ref, prop_ref, dinv_ref, act_ref, propn_ref, qn_ref):
    scat = s_ref[0] + s_ref[1] - q_ref[...]
    dinv = dinv_ref[...]
    xcur = jnp.where(act_ref[...] > 0.5, dinv * scat, prop_ref[...])
    propn_ref[...] = xcur
    qn_ref[...] = dinv * xcur


def _tc_mask(s, q, prop, dinv, act):
    rb = pl.BlockSpec((_RB, _CP), lambda i: (i, 0))
    cb = pl.BlockSpec((_RB, 1), lambda i: (i, 0))
    return pl.pallas_call(
        _mask_body,
        grid=(_NP // _RB,),
        in_specs=[pl.BlockSpec((_NC, _RB, _CP), lambda i: (0, i, 0)),
                  rb, rb, cb, cb],
        out_specs=[rb, rb],
        out_shape=[jax.ShapeDtypeStruct((_NP, _CP), jnp.float32)] * 2,
    )(s, q, prop, dinv, act)


def _heads_body(last, stepval, prop_ref, act_ref, steps_ref, hlp_ref, hv_ref, hent_ref,
                noise_ref, u_ref,
                pW1, pb1, pW2, pb2, pW3, pb3, vW1, vb1, vW2, vb2, vW3, vb3,
                *out_refs):
    w = {"pW1": pW1, "pb1": pb1, "pW2": pW2, "pb2": pb2, "pW3": pW3, "pb3": pb3,
         "vW1": vW1, "vb1": vb1, "vW2": vW2, "vb2": vb2, "vW3": vW3, "vb3": vb3}
    active = act_ref[...] > 0.5
    xcur = prop_ref[...]
    hl, v = _heads(xcur, w)
    p = jax.nn.sigmoid(hl)
    entropy = -(p * jnp.log(p + 1e-10) + (1.0 - p) * jnp.log(1.0 - p + 1e-10))
    noisy_p = jnp.clip(p + noise_ref[0], 0.01, 0.99)
    halt = active & (u_ref[0] < noisy_p)
    lnp = jnp.log(noisy_p + 1e-10)
    hlp = jnp.where(halt, lnp, hlp_ref[...])
    hv = jnp.where(halt, v, hv_ref[...])
    hent = jnp.where(halt, entropy, hent_ref[...])
    active2 = active & (~halt)
    steps = jnp.where(active2, stepval, steps_ref[...])
    if not last:
        (act_o, steps_o, hlp_o, hv_o, hent_o) = out_refs
        act_o[...] = jnp.where(active2, 1.0, 0.0)
        steps_o[...] = steps
        hlp_o[...] = hlp
        hv_o[...] = hv
        hent_o[...] = hent
    else:
        (out_ref, steps_o, hlp_o, hv_o, hent_o) = out_refs
        hlp_o[...] = jnp.where(active2, lnp, hlp)
        hv_o[...] = jnp.where(active2, v, hv)
        hent_o[...] = jnp.where(active2, entropy, hent)
        steps_o[...] = jnp.where(active2, float(_NITER), steps)
        logits = xcur
        m = jnp.max(logits, axis=1, keepdims=True)
        sh = logits - m
        out_ref[...] = sh - jnp.log(jnp.sum(jnp.exp(sh), axis=1, keepdims=True))


def _tc_heads(last, t, prop, act, steps, hlp, hv, hent, noise_cols, u_cols, wts):
    rb = pl.BlockSpec((_RB, _CP), lambda i: (i, 0))
    cb = pl.BlockSpec((_RB, 1), lambda i: (i, 0))
    tcol = pl.BlockSpec((1, _RB, 1), lambda i, t=t: (t, i, 0))
    full = lambda a: pl.BlockSpec(a.shape, lambda i: tuple(0 for _ in a.shape))
    in_specs = [rb, cb, cb, cb, cb, cb, tcol, tcol] + [full(w) for w in wts]
    if not last:
        out_specs = [cb] * 5
        out_shape = [jax.ShapeDtypeStruct((_NP, 1), jnp.float32)] * 5
    else:
        out_specs = [pl.BlockSpec((_RB, 40), lambda i: (i, 0))] + [cb] * 4
        out_shape = [jax.ShapeDtypeStruct((_NP, 40), jnp.float32)] + \
                    [jax.ShapeDtypeStruct((_NP, 1), jnp.float32)] * 4
    import functools
    return pl.pallas_call(
        functools.partial(_heads_body, last, float(t + 2)),
        grid=(_NP // _RB,),
        in_specs=in_specs,
        out_specs=out_specs,
        out_shape=out_shape,
    )(prop, act, steps, hlp, hv, hent, noise_cols, u_cols, *wts)


def kernel(x, edge_index, W1, b1, W2, b2, pW1, pb1, pW2, pb2, pW3, pb3,
           vW1, vb1, vW2, vb2, vW3, vb3):
    f32 = jnp.float32
    W2p = jnp.pad(W2, ((0, 0), (0, _CP - 40)))
    b2p = jnp.pad(b2, (0, _CP - 40)).reshape(1, _CP)
    pW1p = jnp.pad(pW1, ((0, _CP - 40), (0, 0)))
    vW1p = jnp.pad(vW1, ((0, _CP - 40), (0, 0)))
    wts = [pW1p, pb1.reshape(1, -1), pW2, pb2.reshape(1, -1), pW3,
           pb3.reshape(1, 1), vW1p, vb1.reshape(1, -1), vW2, vb2.reshape(1, -1),
           vW3, vb3.reshape(1, 1)]
    src_t = edge_index[0].reshape(_NW, _NBLK, _BLK)
    dst_t = edge_index[1].reshape(_NW, _NBLK, _BLK)

    rkey = jax.random.key(42)
    noise_all = jnp.stack([
        jax.random.normal(jax.random.fold_in(rkey, 2 * t), (_N,), dtype=f32) * _EXPL
        for t in range(_NITER)])
    u_all = jnp.stack([
        jax.random.uniform(jax.random.fold_in(rkey, 2 * t + 1), (_N,), dtype=f32)
        for t in range(_NITER)])
    noise_cols = jnp.pad(noise_all, ((0, 0), (0, _NP - _N)))[:, :, None]
    u_cols = jnp.pad(u_all, ((0, 0), (0, _NP - _N)))[:, :, None]
    xp = jnp.pad(x, ((0, _NP - _N), (0, 0)))

    z = _tc_encoder(xp, W1, b1.reshape(1, -1), W2p, b2p)
    s_deg = _sc_propagate(jnp.ones((_NP, _CP), f32), src_t, dst_t)
    dinv, q = _tc_prep(s_deg, z)

    prop = z
    act = jnp.ones((_NP, 1), f32)
    steps = jnp.ones((_NP, 1), f32)
    hlp = jnp.zeros((_NP, 1), f32)
    hv = jnp.zeros((_NP, 1), f32)
    hent = jnp.zeros((_NP, 1), f32)

    for t in range(_NITER):
        s = _sc_propagate(q, src_t, dst_t)
        prop, q = _tc_mask(s, q, prop, dinv, act)
        last = t == _NITER - 1
        outs = _tc_heads(last, t, prop, act, steps, hlp, hv, hent,
                         noise_cols, u_cols, wts)
        if not last:
            act, steps, hlp, hv, hent = outs
        else:
            out, steps, hlp, hv, hent = outs

    return (out[:_N], steps[:_N].reshape(_N), hlp[:_N].reshape(_N),
            hv[:_N].reshape(_N), hent[:_N].reshape(_N))
